# baseline (speedup 1.0000x reference)
import functools

import jax
import jax.numpy as jnp
from jax import lax
from jax.experimental import pallas as pl
from jax.experimental.pallas import tpu as pltpu
from jax.experimental.pallas import tpu_sc as plsc

D_MODEL = 1024
FFN = 3584
N_EXP = 8
TOKENS = 2048
BLK = 128
NBLK = 40
G = NBLK * BLK

NW = 32
TOK_W = TOKENS // NW
CHUNK = 16


@functools.cache
def _sc_mesh():
    return plsc.VectorSubcoreMesh(core_axis_name="c", subcore_axis_name="s")


def _router_body(x_ref, wg_ref, pos0_ref, pos1_ref, w1_ref, w2_ref,
                 nb_ref, ast_ref, xbf_ref, c1_ref, c2_ref):
    x = x_ref[...]
    xbf_ref[...] = x.astype(jnp.bfloat16)
    wg = wg_ref[...]
    logits = lax.dot_general(x, wg, (((1,), (1,)), ((), ())),
                             preferred_element_type=jnp.float32)
    ids = lax.broadcasted_iota(jnp.int32, logits.shape, 1)
    m1 = jnp.max(logits, axis=1, keepdims=True)
    a1 = jnp.min(jnp.where(logits == m1, ids, N_EXP), axis=1, keepdims=True)
    masked = jnp.where(ids == a1, -jnp.inf, logits)
    m2 = jnp.max(masked, axis=1, keepdims=True)
    a2 = jnp.min(jnp.where(masked == m2, ids, N_EXP), axis=1, keepdims=True)
    w1_ref[...] = jax.nn.sigmoid(m1 - m2)
    w2_ref[...] = jax.nn.sigmoid(m2 - m1)

    oh1 = (ids == a1).astype(jnp.float32)
    oh2 = (ids == a2).astype(jnp.float32)
    rci = lax.broadcasted_iota(jnp.int32, (BLK, BLK), 0)
    cci = lax.broadcasted_iota(jnp.int32, (BLK, BLK), 1)
    texc = (rci > cci).astype(jnp.float32)
    base1 = jnp.zeros((1, N_EXP), jnp.float32)
    base2 = jnp.zeros((1, N_EXP), jnp.float32)
    for blk in range(TOKENS // BLK):
        sl = slice(blk * BLK, (blk + 1) * BLK)
        seg1 = oh1[sl, :]
        seg2 = oh2[sl, :]
        c1_ref[sl, :] = lax.dot_general(
            texc, seg1, (((1,), (0,)), ((), ())),
            preferred_element_type=jnp.float32) + base1
        c2_ref[sl, :] = lax.dot_general(
            texc, seg2, (((1,), (0,)), ((), ())),
            preferred_element_type=jnp.float32) + base2
        base1 = base1 + jnp.sum(seg1, axis=0, keepdims=True)
        base2 = base2 + jnp.sum(seg2, axis=0, keepdims=True)
    total1 = base1
    counts = base1 + base2
    nb = jnp.floor((counts + (BLK - 1)) * (1.0 / BLK))
    e8r = lax.broadcasted_iota(jnp.int32, (N_EXP, N_EXP), 0)
    e8c = lax.broadcasted_iota(jnp.int32, (N_EXP, N_EXP), 1)
    texc8 = (e8r < e8c).astype(jnp.float32)
    astart = lax.dot_general(nb, texc8, (((1,), (0,)), ((), ())),
                             preferred_element_type=jnp.float32) * BLK
    pos0 = jnp.sum(oh1 * (astart + c1_ref[...]), axis=1, keepdims=True)
    pos1 = jnp.sum(oh2 * (astart + total1 + c2_ref[...]), axis=1, keepdims=True)
    pos0_ref[...] = pos0.astype(jnp.int32)
    pos1_ref[...] = pos1.astype(jnp.int32)
    sel = (e8c == e8r).astype(jnp.float32)
    nb_ref[...] = jnp.sum(sel * nb, axis=1, keepdims=True).astype(jnp.int32)
    ast_ref[...] = jnp.sum(sel * astart, axis=1, keepdims=True).astype(jnp.int32)


def _router(x, Wg):
    return pl.pallas_call(
        _router_body,
        grid=(1,),
        in_specs=[pl.BlockSpec((TOKENS, D_MODEL), lambda b: (0, 0)),
                  pl.BlockSpec((N_EXP, D_MODEL), lambda b: (0, 0))],
        out_specs=[pl.BlockSpec((TOKENS, 1), lambda b: (0, 0)),
                   pl.BlockSpec((TOKENS, 1), lambda b: (0, 0)),
                   pl.BlockSpec((TOKENS, 1), lambda b: (0, 0)),
                   pl.BlockSpec((TOKENS, 1), lambda b: (0, 0)),
                   pl.BlockSpec((N_EXP, 1), lambda b: (0, 0)),
                   pl.BlockSpec((N_EXP, 1), lambda b: (0, 0)),
                   pl.BlockSpec((TOKENS, D_MODEL), lambda b: (0, 0))],
        out_shape=[jax.ShapeDtypeStruct((TOKENS, 1), jnp.int32),
                   jax.ShapeDtypeStruct((TOKENS, 1), jnp.int32),
                   jax.ShapeDtypeStruct((TOKENS, 1), jnp.float32),
                   jax.ShapeDtypeStruct((TOKENS, 1), jnp.float32),
                   jax.ShapeDtypeStruct((N_EXP, 1), jnp.int32),
                   jax.ShapeDtypeStruct((N_EXP, 1), jnp.int32),
                   jax.ShapeDtypeStruct((TOKENS, D_MODEL), jnp.bfloat16)],
        scratch_shapes=[pltpu.VMEM((TOKENS, N_EXP), jnp.float32),
                        pltpu.VMEM((TOKENS, N_EXP), jnp.float32)],
    )(x, Wg)


@functools.cache
def _dispatch_kernel():
    @functools.partial(
        pl.kernel, mesh=_sc_mesh(),
        out_type=jax.ShapeDtypeStruct((G, D_MODEL // 2), jnp.int32),
        scratch_types=[pltpu.VMEM((TOK_W,), jnp.int32),
                       pltpu.VMEM((TOK_W,), jnp.int32),
                       pltpu.VMEM((TOK_W, D_MODEL // 2), jnp.int32),
                       pltpu.SemaphoreType.DMA,
                       pltpu.SemaphoreType.DMA])
    def _dispatch(xbf_hbm, pos0_hbm, pos1_hbm, xs_hbm, idx0_v, idx1_v, rows_v,
                  sem0, sem1):
        wid = lax.axis_index("s") * 2 + lax.axis_index("c")
        t0 = wid * TOK_W
        pltpu.sync_copy(xbf_hbm.at[pl.ds(t0, TOK_W)], rows_v)
        pltpu.sync_copy(pos0_hbm.at[pl.ds(t0, TOK_W)], idx0_v)
        pltpu.sync_copy(pos1_hbm.at[pl.ds(t0, TOK_W)], idx1_v)
        cp0 = pltpu.async_copy(rows_v, xs_hbm.at[idx0_v], sem0)
        cp1 = pltpu.async_copy(rows_v, xs_hbm.at[idx1_v], sem1)
        cp0.wait()
        cp1.wait()

    return _dispatch


NF = 7
FT = FFN // NF


def _ffn_body(nb_ref, ast_ref, xs_ref, w1_ref, w3_ref, w2_ref, o_ref):
    e = pl.program_id(0)
    f = pl.program_id(1)
    w1 = w1_ref[0].astype(jnp.bfloat16)
    w3 = w3_ref[0].astype(jnp.bfloat16)
    w2 = w2_ref[0].astype(jnp.bfloat16)
    n_blocks = nb_ref[e]
    r0 = ast_ref[e]

    def body(i, carry):
        off = pl.multiple_of(r0 + i * BLK, BLK)
        xb = xs_ref[pl.ds(off, BLK), :]
        a = lax.dot_general(xb, w1, (((1,), (1,)), ((), ())),
                            preferred_element_type=jnp.float32)
        b = lax.dot_general(xb, w3, (((1,), (1,)), ((), ())),
                            preferred_element_type=jnp.float32)
        h = (a * jax.nn.sigmoid(a) * b).astype(jnp.bfloat16)
        yp = lax.dot_general(h, w2, (((1,), (1,)), ((), ())),
                             preferred_element_type=jnp.float32)

        @pl.when(f == 0)
        def _init():
            o_ref[pl.ds(off, BLK), :] = yp

        @pl.when(f != 0)
        def _acc():
            o_ref[pl.ds(off, BLK), :] += yp

        return carry

    lax.fori_loop(0, n_blocks, body, 0)


def _ffn(nb8, ast8, xs, W1, W3, W2):
    grid_spec = pltpu.PrefetchScalarGridSpec(
        num_scalar_prefetch=2,
        grid=(N_EXP, NF),
        in_specs=[
            pl.BlockSpec((G, D_MODEL), lambda e, f, nb, ast: (0, 0)),
            pl.BlockSpec((1, FT, D_MODEL), lambda e, f, nb, ast: (e, f, 0)),
            pl.BlockSpec((1, FT, D_MODEL), lambda e, f, nb, ast: (e, f, 0)),
            pl.BlockSpec((1, D_MODEL, FT), lambda e, f, nb, ast: (e, 0, f)),
        ],
        out_specs=pl.BlockSpec((G, D_MODEL), lambda e, f, nb, ast: (0, 0)),
    )
    return pl.pallas_call(
        _ffn_body,
        grid_spec=grid_spec,
        out_shape=jax.ShapeDtypeStruct((G, D_MODEL), jnp.float32),
    )(nb8, ast8, xs, W1, W3, W2)


@functools.cache
def _combine_kernel():
    @functools.partial(
        pl.kernel, mesh=_sc_mesh(),
        out_type=jax.ShapeDtypeStruct((TOKENS, D_MODEL), jnp.float32),
        scratch_types=[pltpu.VMEM((TOK_W,), jnp.int32),
                       pltpu.VMEM((TOK_W,), jnp.int32),
                       pltpu.VMEM((TOK_W,), jnp.float32),
                       pltpu.VMEM((TOK_W,), jnp.float32),
                       pltpu.VMEM((2, CHUNK, D_MODEL), jnp.float32),
                       pltpu.VMEM((2, CHUNK, D_MODEL), jnp.float32),
                       pltpu.SemaphoreType.DMA,
                       pltpu.SemaphoreType.DMA,
                       pltpu.SemaphoreType.DMA,
                       pltpu.SemaphoreType.DMA])
    def _combine(ys_hbm, pos0_hbm, pos1_hbm, w0_hbm, w1_hbm, out_hbm,
                 p0_v, p1_v, w0_v, w1_v, a0_v, a1_v, s00, s01, s10, s11):
        wid = lax.axis_index("s") * 2 + lax.axis_index("c")
        t0 = wid * TOK_W
        nch = TOK_W // CHUNK
        sems0 = (s00, s01)
        sems1 = (s10, s11)
        pltpu.sync_copy(pos0_hbm.at[pl.ds(t0, TOK_W)], p0_v)
        pltpu.sync_copy(pos1_hbm.at[pl.ds(t0, TOK_W)], p1_v)
        pltpu.sync_copy(w0_hbm.at[pl.ds(t0, TOK_W)], w0_v)
        pltpu.sync_copy(w1_hbm.at[pl.ds(t0, TOK_W)], w1_v)

        def issue(chunk, par):
            i0 = p0_v[pl.ds(chunk * CHUNK, CHUNK)]
            i1 = p1_v[pl.ds(chunk * CHUNK, CHUNK)]
            return (pltpu.async_copy(ys_hbm.at[i0], a0_v.at[par], sems0[par]),
                    pltpu.async_copy(ys_hbm.at[i1], a1_v.at[par], sems1[par]))

        cps = issue(0, 0)
        for chunk in range(nch):
            par = chunk & 1
            nxt = issue(chunk + 1, 1 - par) if chunk + 1 < nch else None
            cps[0].wait()
            cps[1].wait()
            wv0 = w0_v[pl.ds(chunk * CHUNK, CHUNK)]
            wv1 = w1_v[pl.ds(chunk * CHUNK, CHUNK)]
            for li in range(CHUNK):
                w0 = wv0[li]
                w1 = wv1[li]

                @pl.loop(0, D_MODEL // 16, unroll=4)
                def _col(c):
                    sl = pl.ds(c * 16, 16)
                    a0_v[par, li, sl] = (w0 * a0_v[par, li, sl]
                                         + w1 * a1_v[par, li, sl])

            pltpu.sync_copy(a0_v.at[par],
                            out_hbm.at[pl.ds(t0 + chunk * CHUNK, CHUNK)])
            cps = nxt

    return _combine


def kernel(hidden_states, Wg, W1, W3, W2):
    b, s, d = hidden_states.shape
    x = hidden_states.reshape(TOKENS, D_MODEL)
    pos0, pos1, w1f, w2f, nb8, ast8, xbf = _router(x, Wg)
    pos0 = pos0[:, 0]
    pos1 = pos1[:, 0]
    xbf_i32 = lax.bitcast_convert_type(
        xbf.reshape(TOKENS, D_MODEL // 2, 2), jnp.int32)
    xs_i32 = _dispatch_kernel()(xbf_i32, pos0, pos1)
    xs = lax.bitcast_convert_type(xs_i32, jnp.bfloat16).reshape(G, D_MODEL)
    ys = _ffn(nb8[:, 0], ast8[:, 0], xs, W1, W3, W2)
    out = _combine_kernel()(ys, pos0, pos1, w1f[:, 0], w2f[:, 0])
    return out.reshape(b, s, d)
